# Initial kernel scaffold; baseline (speedup 1.0000x reference)
import jax
import jax.numpy as jnp
from jax.experimental import pallas as pl


def kernel(segments, table):
    raise NotImplementedError("write your pallas kernel here")


# speedup vs baseline: 5.1895x; 5.1895x over previous
import jax
import jax.numpy as jnp
from jax.experimental import pallas as pl
from jax.experimental.pallas import tpu as pltpu

_HID = 1024
_BLK = 1024


def _select_body(seg_ref, tab_ref, out_ref):
    seg = seg_ref[0, 0, :]
    segf = seg.astype(jnp.float32)[:, None]
    t0 = tab_ref[0, :][None, :]
    t1 = tab_ref[1, :][None, :]
    out_ref[...] = t0 + segf * (t1 - t0)


def kernel(segments, table):
    batch, seq = segments.shape
    n = batch * seq
    nblk = n // _BLK
    seg3 = segments.reshape(nblk, 1, _BLK).astype(jnp.int32)

    out = pl.pallas_call(
        _select_body,
        grid=(nblk,),
        in_specs=[
            pl.BlockSpec((1, 1, _BLK), lambda i: (i, 0, 0)),
            pl.BlockSpec((2, _HID), lambda i: (0, 0)),
        ],
        out_specs=pl.BlockSpec((_BLK, _HID), lambda i: (i, 0)),
        out_shape=jax.ShapeDtypeStruct((n, _HID), jnp.float32),
    )(seg3, table)
    return out.reshape(batch, seq, _HID)
